# baseline (speedup 1.0000x reference)
import functools

import jax
import jax.numpy as jnp
from jax import lax
from jax.experimental import pallas as pl
from jax.experimental.pallas import tpu as pltpu
from jax.experimental.pallas import tpu_sc as plsc

B = 4
T = 16384
HID = 128
C_DIM = 128
RESO = 64
S = RESO * RESO
PAD = 0.1
NB = 5

NCORE = 2
NSUB = 16
LANES = 16

TPB = T // NSUB
ROWS = T // 128
TAB = 3 * S
DEN = 1.0 + PAD + 1e-3

_mesh = plsc.VectorSubcoreMesh(core_axis_name="c", subcore_axis_name="s")


def _digitize(v):
    u = jnp.clip(v / DEN + 0.5, 0.0, 1.0 - 1e-6)
    return jnp.clip((u * RESO).astype(jnp.int32), 0, RESO - 1)


def _idx_body(pt_hbm, ones_hbm, zc_hbm, idxoff_hbm, inv_hbm,
              px_v, py_v, pz_v, ioff_v, coff_v, ones_v, cnt_v, inv_v, counts_sh):
    c = lax.axis_index("c")
    s = lax.axis_index("s")
    b_local = s // 8
    b = 2 * c + b_local
    base = (s % 8) * 2048

    sh0 = pl.multiple_of(s * 1536, 8)
    pltpu.sync_copy(zc_hbm.at[pl.ds(sh0, 1536)],
                    counts_sh.at[pl.ds(sh0, 1536)])
    pltpu.sync_copy(ones_hbm, ones_v)

    prow = b * 3 * 128 + (s % 8) * 16
    pltpu.sync_copy(pt_hbm.at[pl.ds(pl.multiple_of(prow, 8), 16)], px_v)
    pltpu.sync_copy(pt_hbm.at[pl.ds(pl.multiple_of(prow + 128, 8), 16)], py_v)
    pltpu.sync_copy(pt_hbm.at[pl.ds(pl.multiple_of(prow + 256, 8), 16)], pz_v)

    boff = b_local * TAB

    def step(i, _):
        prw = i // 8
        pcl = pl.ds((i % 8) * 16, 16)
        ix = _digitize(px_v[prw, pcl])
        iy = _digitize(py_v[prw, pcl])
        iz = _digitize(pz_v[prw, pcl])
        row = i // 4
        colsl = pl.ds((i % 4) * 16, 16)
        v0 = ix + 64 * iz
        v1 = ix + 64 * iy + S
        v2 = iy + 64 * iz + 2 * S
        ioff_v[row, colsl] = v0
        ioff_v[32 + row, colsl] = v1
        ioff_v[64 + row, colsl] = v2
        coff_v[row, colsl] = v0 + boff
        coff_v[32 + row, colsl] = v1 + boff
        coff_v[64 + row, colsl] = v2 + boff
        return _

    lax.fori_loop(0, 128, step, None)

    for plane in range(3):
        r0 = pl.multiple_of((b * 3 + plane) * (T // 64) + base // 64, 8)
        pltpu.sync_copy(ioff_v.at[pl.ds(plane * 32, 32)],
                        idxoff_hbm.at[pl.ds(r0, 32)])

    plsc.subcore_barrier()
    for r in range(96):
        pltpu.sync_copy(ones_v, counts_sh.at[coff_v.at[r]], add=True)
    plsc.subcore_barrier()

    pltpu.sync_copy(counts_sh.at[pl.ds(sh0, 1536)], cnt_v)

    def invstep(r, _):
        inv_v[r] = 1.0 / jnp.maximum(cnt_v[r], 1.0)
        return _

    lax.fori_loop(0, 1536, invstep, None)
    pltpu.sync_copy(inv_v, inv_hbm.at[pl.ds(pl.multiple_of(c * 2 * TAB + s * 1536, 8), 1536)])


_SC_PARAMS = pltpu.CompilerParams(use_tc_tiling_on_sc=False)

_idx_call = functools.partial(
    pl.kernel,
    compiler_params=_SC_PARAMS,
    out_type=(
        jax.ShapeDtypeStruct((B * 3 * (T // 64), 64), jnp.int32),
        jax.ShapeDtypeStruct((B * TAB, 16), jnp.float32),
    ),
    mesh=_mesh,
    scratch_types=[
        pltpu.VMEM((16, 128), jnp.float32),
        pltpu.VMEM((16, 128), jnp.float32),
        pltpu.VMEM((16, 128), jnp.float32),
        pltpu.VMEM((96, 64), jnp.int32),
        pltpu.VMEM((96, 64), jnp.int32),
        pltpu.VMEM((64, 16), jnp.float32),
        pltpu.VMEM((1536, 16), jnp.float32),
        pltpu.VMEM((1536, 16), jnp.float32),
        pltpu.VMEM_SHARED((2 * TAB, 16), jnp.float32),
    ],
)(_idx_body)


def _pool_wave(do_gather, b, c_hbm, dest_hbm, idxoff_hbm, inv_hbm, s, refs):
    if do_gather:
        (off_v, ga, gb, gc, iva, ivb, ivc, s0, s1, s2, sa0, sa1,
         tables_sh) = refs
        inv3_hbm = inv_hbm
    else:
        (off_v, ga, gb, inv_c, inv_c2, s0, s1, s2, sa0, sa1, tables_sh) = refs
    R64 = T // 64

    def zstep(r, _):
        for k in range(8):
            ga[r, pl.ds(k * 16, 16)] = jnp.zeros((16,), jnp.float32)
        return _

    lax.fori_loop(0, 64, zstep, None)
    for j2 in range(12):
        r0 = pl.multiple_of(s * 768 + j2 * 64, 8)
        pltpu.sync_copy(ga, tables_sh.at[pl.ds(r0, 64)])
    for plane in range(3):
        ir0 = pl.multiple_of((b * 3 + plane) * R64 + s * 16, 8)
        pltpu.sync_copy(idxoff_hbm.at[pl.ds(ir0, 16)],
                        off_v.at[pl.ds(plane * 16, 16)])
    plsc.subcore_barrier()

    bufs = (ga, gb)
    lsems = (s0, s1)
    ssems = (sa0, sa1)
    pend_sc = [None, None]

    def chunk_src(j):
        return c_hbm.at[b, pl.ds(pl.multiple_of(s * TPB + j * 64, 8), 64)]

    pend_load = pltpu.async_copy(chunk_src(0), bufs[0], lsems[0])
    for j in range(16):
        pend_load.wait()
        if j < 15:
            nb = (j + 1) % 2
            if pend_sc[nb] is not None:
                for d in pend_sc[nb]:
                    d.wait()
                pend_sc[nb] = None
            pend_load = pltpu.async_copy(chunk_src(j + 1), bufs[nb], lsems[nb])
        pend_sc[j % 2] = [
            pltpu.async_copy(bufs[j % 2],
                             tables_sh.at[off_v.at[plane * 16 + j]],
                             ssems[j % 2], add=True)
            for plane in range(3)
        ]
    for lst in pend_sc:
        if lst is not None:
            for d in lst:
                d.wait()
    plsc.subcore_barrier()

    if do_gather:
        bufs3 = (ga, gb, gc)
        ivbufs = (iva, ivb, ivc)
        gsems = (s0, s1, s2)
        sdesc = [None, None, None]
        for j in range(16):
            bi0 = j % 3
            bi1 = (j + 1) % 3
            bi2 = (j + 2) % 3
            for bi in (bi0, bi1, bi2):
                if sdesc[bi] is not None:
                    sdesc[bi].wait()
                    sdesc[bi] = None
            a, b2, c3 = bufs3[bi0], bufs3[bi1], bufs3[bi2]
            d0 = pltpu.async_copy(tables_sh.at[off_v.at[j]], a, gsems[bi0])
            d1 = pltpu.async_copy(tables_sh.at[off_v.at[16 + j]], b2,
                                  gsems[bi1])
            d2 = pltpu.async_copy(tables_sh.at[off_v.at[32 + j]], c3,
                                  gsems[bi2])
            e0 = pltpu.async_copy(inv3_hbm.at[b].at[off_v.at[j]], iva,
                                  sa1)
            e1 = pltpu.async_copy(inv3_hbm.at[b].at[off_v.at[16 + j]], ivb,
                                  sa1)
            e2 = pltpu.async_copy(inv3_hbm.at[b].at[off_v.at[32 + j]], ivc,
                                  sa1)
            d0.wait()
            d1.wait()
            d2.wait()
            e0.wait()
            e1.wait()
            e2.wait()

            def acc3(r, _, a=a, b2=b2, c3=c3):
                va = iva[r]
                vb = ivb[r]
                vc = ivc[r]
                for k in range(8):
                    sl = pl.ds(k * 16, 16)
                    a[r, sl] = a[r, sl] * va + b2[r, sl] * vb + c3[r, sl] * vc
                return _

            lax.fori_loop(0, 64, acc3, None)
            sdesc[bi0] = pltpu.async_copy(
                a,
                dest_hbm.at[b, pl.ds(pl.multiple_of(s * TPB + j * 64, 8), 64)],
                sa0)
        for bi in range(3):
            if sdesc[bi] is not None:
                sdesc[bi].wait()
        plsc.subcore_barrier()
    else:
        nbufs = (ga, gb)
        wsems = (s0, s1)
        ibufs = (inv_c, inv_c2)
        wdesc = [None, None]

        def inv_src(j2):
            gr0 = pl.multiple_of(b * TAB + s * 768 + j2 * 64, 8)
            return inv_hbm.at[pl.ds(gr0, 64)]

        ipend = pltpu.async_copy(inv_src(0), ibufs[0], s2)
        for j2 in range(12):
            nb = j2 % 2
            if wdesc[nb] is not None:
                wdesc[nb].wait()
                wdesc[nb] = None
            buf = nbufs[nb]
            ib = ibufs[nb]
            r0 = pl.multiple_of(s * 768 + j2 * 64, 8)
            gr0 = pl.multiple_of(b * TAB + s * 768 + j2 * 64, 8)
            pltpu.sync_copy(tables_sh.at[pl.ds(r0, 64)], buf)
            ipend.wait()
            if j2 < 11:
                ipend = pltpu.async_copy(inv_src(j2 + 1), ibufs[(j2 + 1) % 2], s2)

            def nstep(r, _, buf=buf, ib=ib):
                iv = ib[r]
                for k in range(8):
                    sl = pl.ds(k * 16, 16)
                    buf[r, sl] = buf[r, sl] * iv
                return _

            lax.fori_loop(0, 64, nstep, None)
            wdesc[nb] = pltpu.async_copy(buf, dest_hbm.at[pl.ds(gr0, 64)],
                                         wsems[nb])
        for i2 in range(2):
            if wdesc[i2] is not None:
                wdesc[i2].wait()
        plsc.subcore_barrier()


def _pool1_body(do_gather, c_hbm, idxoff_hbm, inv_hbm, *refs):
    out1 = refs[0]
    rest = refs[1:]
    c = lax.axis_index("c")
    s = lax.axis_index("s")
    for w in range(2):
        b = 2 * c + w
        _pool_wave(do_gather, b, c_hbm, out1, idxoff_hbm, inv_hbm, s, rest)


def _make_pool(do_gather):
    if do_gather:
        one = jax.ShapeDtypeStruct((B, T, 128), jnp.float32)
        mid = [
            pltpu.VMEM((64, 128), jnp.float32),
            pltpu.VMEM((64, 128), jnp.float32),
            pltpu.VMEM((64, 128), jnp.float32),
            pltpu.VMEM((64, 16), jnp.float32),
            pltpu.VMEM((64, 16), jnp.float32),
            pltpu.VMEM((64, 16), jnp.float32),
        ]
    else:
        one = jax.ShapeDtypeStruct((B * TAB, 128), jnp.float32)
        mid = [
            pltpu.VMEM((64, 128), jnp.float32),
            pltpu.VMEM((64, 128), jnp.float32),
            pltpu.VMEM((64, 16), jnp.float32),
            pltpu.VMEM((64, 16), jnp.float32),
        ]
    scratch = (
        [pltpu.VMEM((48, 64), jnp.int32)]
        + mid
        + [
            pltpu.SemaphoreType.DMA,
            pltpu.SemaphoreType.DMA,
            pltpu.SemaphoreType.DMA,
            pltpu.SemaphoreType.DMA,
            pltpu.SemaphoreType.DMA,
            pltpu.VMEM_SHARED((TAB, 128), jnp.float32),
        ]
    )
    return pl.kernel(
        functools.partial(_pool1_body, do_gather),
        out_type=one,
        mesh=_mesh,
        scratch_types=scratch,
        compiler_params=_SC_PARAMS,
    )


_pool_call = _make_pool(True)
_plane_call = _make_pool(False)


RE = 2048
BT = B * T


def _full(shape):
    return pl.BlockSpec(shape, lambda i: (0,) * len(shape))


def _rows(shape):
    return pl.BlockSpec(shape, lambda i: (i,) + (0,) * (len(shape) - 1))


def _dot(a, b):
    return jnp.dot(a, b, preferred_element_type=jnp.float32)


def _embed_body(p_ref, p2_ref, wp, bp, wp2, bp2, wfa, wfb, bf, fp_ref, nc_ref):
    x = p_ref[...]
    x2 = p2_ref[...]
    fp = jnp.maximum(_dot(x, wp[...]) + bp[...], 0.0)
    fp2 = jnp.maximum(_dot(x2, wp2[...]) + bp2[...], 0.0)
    nc = jnp.maximum(_dot(fp, wfa[...]) + _dot(fp2, wfb[...]) + bf[...], 0.0)
    fp_ref[...] = fp
    nc_ref[...] = nc


_embed_call = pl.pallas_call(
    _embed_body,
    grid=(BT // RE,),
    in_specs=[
        _rows((RE, 3)), _rows((RE, 3)),
        _full((3, 2 * HID)), _full((1, 2 * HID)),
        _full((3, 2 * HID)), _full((1, 2 * HID)),
        _full((2 * HID, 2 * HID)), _full((2 * HID, 2 * HID)),
        _full((1, 2 * HID)),
    ],
    out_specs=(_rows((RE, 2 * HID)), _rows((RE, 2 * HID))),
    out_shape=(
        jax.ShapeDtypeStruct((BT, 2 * HID), jnp.float32),
        jax.ShapeDtypeStruct((BT, 2 * HID), jnp.float32),
    ),
)


def _res1_body(x_ref, w0, b0, w1, b1, ws, o_ref):
    x = x_ref[...]
    net = _dot(jnp.maximum(x, 0.0), w0[...]) + b0[...]
    dx = _dot(jnp.maximum(net, 0.0), w1[...]) + b1[...]
    o_ref[...] = _dot(x, ws[...]) + dx


_res1_call = pl.pallas_call(
    _res1_body,
    grid=(BT // RE,),
    in_specs=[
        _rows((RE, 2 * HID)),
        _full((2 * HID, HID)), _full((1, HID)),
        _full((HID, HID)), _full((1, HID)),
        _full((2 * HID, HID)),
    ],
    out_specs=_rows((RE, HID)),
    out_shape=jax.ShapeDtypeStruct((BT, HID), jnp.float32),
)


def _res2_body(has_proj, *refs):
    if has_proj:
        (a_ref, p_ref, w0a, w0b, b0, w1, b1, wsa, wsb, wc, bc, o_ref) = refs
    else:
        (a_ref, p_ref, w0a, w0b, b0, w1, b1, wsa, wsb, o_ref) = refs
    a = a_ref[...]
    p = p_ref[...]
    net = (_dot(jnp.maximum(a, 0.0), w0a[...])
           + _dot(jnp.maximum(p, 0.0), w0b[...]) + b0[...])
    dx = _dot(jnp.maximum(net, 0.0), w1[...]) + b1[...]
    o = _dot(a, wsa[...]) + _dot(p, wsb[...]) + dx
    if has_proj:
        o = _dot(o, wc[...]) + bc[...]
    o_ref[...] = o


def _make_res2(has_proj, mout):
    in_specs = [
        _rows((RE, HID)), _rows((RE, HID)),
        _full((HID, HID)), _full((HID, HID)), _full((1, HID)),
        _full((HID, HID)), _full((1, HID)),
        _full((HID, HID)), _full((HID, HID)),
    ]
    if has_proj:
        in_specs += [_full((HID, mout)), _full((1, mout))]
    return pl.pallas_call(
        functools.partial(_res2_body, has_proj),
        grid=(BT // RE,),
        in_specs=in_specs,
        out_specs=_rows((RE, mout)),
        out_shape=jax.ShapeDtypeStruct((BT, mout), jnp.float32),
    )


_res2_call = _make_res2(False, HID)
_res2p_call = _make_res2(True, C_DIM)


def _res1(x, blk):
    return _res1_call(x, blk['W0'], blk['b0'].reshape(1, -1),
                      blk['W1'], blk['b1'].reshape(1, -1), blk['Ws'])


def _res2(net, pooled, blk, proj=None):
    args = (net, pooled,
            blk['W0'][:HID], blk['W0'][HID:], blk['b0'].reshape(1, -1),
            blk['W1'], blk['b1'].reshape(1, -1),
            blk['Ws'][:HID], blk['Ws'][HID:])
    if proj is None:
        return _res2_call(*args)
    wc, bc = proj
    return _res2p_call(*args, wc, bc.reshape(1, -1))


def kernel(p, p2, params):
    pf = p.reshape(BT, 3)
    p2f = p2.reshape(BT, 3)
    prm = params
    fp, ncorr = _embed_call(
        pf, p2f, prm['Wp'], prm['bp'].reshape(1, -1),
        prm['Wp2'], prm['bp2'].reshape(1, -1),
        prm['Wf'][:2 * HID], prm['Wf'][2 * HID:], prm['bf'].reshape(1, -1))

    pt = jnp.transpose(p, (0, 2, 1)).reshape(B * 3 * 128, 128)
    ones = jnp.ones((64, 16), jnp.float32)
    zc = jnp.zeros((2 * TAB, 16), jnp.float32)
    idxoff, inv = _idx_call(pt, ones, zc)

    blocks_g = prm['blocks']
    blocks_c = prm['blocks_corr']
    net_g = _res1(fp, blocks_g[0])
    net_c = _res1(ncorr, blocks_c[0])
    nblk = len(blocks_g)
    inv3 = inv.reshape(B, TAB, 16)
    for i in range(1, nblk):
        pooled_g = _pool_call(net_g.reshape(B, T, HID), idxoff, inv3)
        pooled_c = _pool_call(net_c.reshape(B, T, HID), idxoff, inv3)
        last = i == nblk - 1
        net_g = _res2(net_g, pooled_g.reshape(BT, HID), blocks_g[i],
                      (prm['Wc'], prm['bc']) if last else None)
        net_c = _res2(net_c, pooled_c.reshape(BT, HID), blocks_c[i],
                      (prm['Wc_corr'], prm['bc_corr']) if last else None)

    tabs_c = _plane_call(net_g.reshape(B, T, C_DIM), idxoff, inv)
    tabs_cc = _plane_call(net_c.reshape(B, T, C_DIM), idxoff, inv)

    def to_fea(tabs):
        f = tabs.reshape(B, 3, S, C_DIM)
        f = jnp.transpose(f, (1, 0, 3, 2))
        return f.reshape(3, B, C_DIM, RESO, RESO)

    return jnp.concatenate([to_fea(tabs_c), to_fea(tabs_cc)], axis=0)


# speedup vs baseline: 1.3091x; 1.3091x over previous
import functools

import jax
import jax.numpy as jnp
from jax import lax
from jax.experimental import pallas as pl
from jax.experimental.pallas import tpu as pltpu
from jax.experimental.pallas import tpu_sc as plsc

B = 4
T = 16384
HID = 128
C_DIM = 128
RESO = 64
S = RESO * RESO
PAD = 0.1
NB = 5

NCORE = 2
NSUB = 16
LANES = 16

TPB = T // NSUB
ROWS = T // 128
TAB = 3 * S
DEN = 1.0 + PAD + 1e-3

_mesh = plsc.VectorSubcoreMesh(core_axis_name="c", subcore_axis_name="s")


def _digitize(v):
    u = jnp.clip(v / DEN + 0.5, 0.0, 1.0 - 1e-6)
    return jnp.clip((u * RESO).astype(jnp.int32), 0, RESO - 1)


def _idx_body(pt_hbm, ones_hbm, zc_hbm, idxoff_hbm, inv_hbm,
              px_v, py_v, pz_v, ioff_v, coff_v, ones_v, cnt_v, inv_v, counts_sh):
    c = lax.axis_index("c")
    s = lax.axis_index("s")
    b_local = s // 8
    b = 2 * c + b_local
    base = (s % 8) * 2048

    sh0 = pl.multiple_of(s * 1536, 8)
    pltpu.sync_copy(zc_hbm.at[pl.ds(sh0, 1536)],
                    counts_sh.at[pl.ds(sh0, 1536)])
    pltpu.sync_copy(ones_hbm, ones_v)

    prow = b * 3 * 128 + (s % 8) * 16
    pltpu.sync_copy(pt_hbm.at[pl.ds(pl.multiple_of(prow, 8), 16)], px_v)
    pltpu.sync_copy(pt_hbm.at[pl.ds(pl.multiple_of(prow + 128, 8), 16)], py_v)
    pltpu.sync_copy(pt_hbm.at[pl.ds(pl.multiple_of(prow + 256, 8), 16)], pz_v)

    boff = b_local * TAB

    def step(i, _):
        prw = i // 8
        pcl = pl.ds((i % 8) * 16, 16)
        ix = _digitize(px_v[prw, pcl])
        iy = _digitize(py_v[prw, pcl])
        iz = _digitize(pz_v[prw, pcl])
        row = i // 4
        colsl = pl.ds((i % 4) * 16, 16)
        v0 = ix + 64 * iz
        v1 = ix + 64 * iy + S
        v2 = iy + 64 * iz + 2 * S
        ioff_v[row, colsl] = v0
        ioff_v[32 + row, colsl] = v1
        ioff_v[64 + row, colsl] = v2
        coff_v[row, colsl] = v0 + boff
        coff_v[32 + row, colsl] = v1 + boff
        coff_v[64 + row, colsl] = v2 + boff
        return _

    lax.fori_loop(0, 128, step, None)

    for plane in range(3):
        r0 = pl.multiple_of((b * 3 + plane) * (T // 64) + base // 64, 8)
        pltpu.sync_copy(ioff_v.at[pl.ds(plane * 32, 32)],
                        idxoff_hbm.at[pl.ds(r0, 32)])

    plsc.subcore_barrier()
    for r in range(96):
        pltpu.sync_copy(ones_v, counts_sh.at[coff_v.at[r]], add=True)
    plsc.subcore_barrier()

    pltpu.sync_copy(counts_sh.at[pl.ds(sh0, 1536)], cnt_v)

    def invstep(r, _):
        inv_v[r] = 1.0 / jnp.maximum(cnt_v[r], 1.0)
        return _

    lax.fori_loop(0, 1536, invstep, None)
    pltpu.sync_copy(inv_v, inv_hbm.at[pl.ds(pl.multiple_of(c * 2 * TAB + s * 1536, 8), 1536)])


_SC_PARAMS = pltpu.CompilerParams(use_tc_tiling_on_sc=False)

_idx_call = functools.partial(
    pl.kernel,
    compiler_params=_SC_PARAMS,
    out_type=(
        jax.ShapeDtypeStruct((B * 3 * (T // 64), 64), jnp.int32),
        jax.ShapeDtypeStruct((B * TAB, 16), jnp.float32),
    ),
    mesh=_mesh,
    scratch_types=[
        pltpu.VMEM((16, 128), jnp.float32),
        pltpu.VMEM((16, 128), jnp.float32),
        pltpu.VMEM((16, 128), jnp.float32),
        pltpu.VMEM((96, 64), jnp.int32),
        pltpu.VMEM((96, 64), jnp.int32),
        pltpu.VMEM((64, 16), jnp.float32),
        pltpu.VMEM((1536, 16), jnp.float32),
        pltpu.VMEM((1536, 16), jnp.float32),
        pltpu.VMEM_SHARED((2 * TAB, 16), jnp.float32),
    ],
)(_idx_body)


def _pool_wave(do_gather, b, c_hbm, dest_hbm, idxoff_hbm, inv_hbm, s, refs):
    if do_gather:
        (off_v, ga, gb, gc, inv_c, inv_c2, s0, s1, s2, sa0, sa1,
         tables_sh) = refs
    else:
        (off_v, ga, gb, inv_c, inv_c2, s0, s1, s2, sa0, sa1, tables_sh) = refs
    R64 = T // 64

    def zstep(r, _):
        for k in range(8):
            ga[r, pl.ds(k * 16, 16)] = jnp.zeros((16,), jnp.float32)
        return _

    lax.fori_loop(0, 64, zstep, None)
    zdescs = []
    for j2 in range(12):
        r0 = pl.multiple_of(s * 768 + j2 * 64, 8)
        zdescs.append(pltpu.async_copy(ga, tables_sh.at[pl.ds(r0, 64)], sa0))
    odescs = []
    for plane in range(3):
        ir0 = pl.multiple_of((b * 3 + plane) * R64 + s * 16, 8)
        odescs.append(pltpu.async_copy(idxoff_hbm.at[pl.ds(ir0, 16)],
                                       off_v.at[pl.ds(plane * 16, 16)], sa1))
    for d in zdescs + odescs:
        d.wait()
    plsc.subcore_barrier()

    bufs = (ga, gb)
    lsems = (s0, s1)
    ssems = (sa0, sa1)
    pend_sc = [None, None]

    def chunk_src(j):
        return c_hbm.at[b, pl.ds(pl.multiple_of(s * TPB + j * 64, 8), 64)]

    pend_load = pltpu.async_copy(chunk_src(0), bufs[0], lsems[0])
    for j in range(16):
        pend_load.wait()
        if j < 15:
            nb = (j + 1) % 2
            if pend_sc[nb] is not None:
                for d in pend_sc[nb]:
                    d.wait()
                pend_sc[nb] = None
            pend_load = pltpu.async_copy(chunk_src(j + 1), bufs[nb], lsems[nb])
        pend_sc[j % 2] = [
            pltpu.async_copy(bufs[j % 2],
                             tables_sh.at[off_v.at[plane * 16 + j]],
                             ssems[j % 2], add=True)
            for plane in range(3)
        ]
    for lst in pend_sc:
        if lst is not None:
            for d in lst:
                d.wait()
    plsc.subcore_barrier()

    nbufs = (ga, gb)
    wsems = (s0, s1)
    ibufs = (inv_c, inv_c2)
    wdesc = [None, None]

    def inv_src(j2):
        gr0 = pl.multiple_of(b * TAB + s * 768 + j2 * 64, 8)
        return inv_hbm.at[pl.ds(gr0, 64)]

    ipend = pltpu.async_copy(inv_src(0), ibufs[0], s2)
    for j2 in range(12):
        nb = j2 % 2
        if wdesc[nb] is not None:
            wdesc[nb].wait()
            wdesc[nb] = None
        buf = nbufs[nb]
        ib = ibufs[nb]
        r0 = pl.multiple_of(s * 768 + j2 * 64, 8)
        gr0 = pl.multiple_of(b * TAB + s * 768 + j2 * 64, 8)
        pltpu.sync_copy(tables_sh.at[pl.ds(r0, 64)], buf)
        ipend.wait()
        if j2 < 11:
            ipend = pltpu.async_copy(inv_src(j2 + 1), ibufs[(j2 + 1) % 2], s2)

        def nstep(r, _, buf=buf, ib=ib):
            iv = ib[r]
            for k in range(8):
                sl = pl.ds(k * 16, 16)
                buf[r, sl] = buf[r, sl] * iv
            return _

        lax.fori_loop(0, 64, nstep, None)
        if do_gather:
            wdesc[nb] = pltpu.async_copy(buf, tables_sh.at[pl.ds(r0, 64)],
                                         wsems[nb])
        else:
            wdesc[nb] = pltpu.async_copy(buf, dest_hbm.at[pl.ds(gr0, 64)],
                                         wsems[nb])
    for i2 in range(2):
        if wdesc[i2] is not None:
            wdesc[i2].wait()
    plsc.subcore_barrier()

    if do_gather:
        bufs3 = (ga, gb, gc)
        gsems = (s0, s1, s2)
        sdesc = [None, None, None]
        for j in range(16):
            bi0 = j % 3
            bi1 = (j + 1) % 3
            bi2 = (j + 2) % 3
            for bi in (bi0, bi1, bi2):
                if sdesc[bi] is not None:
                    sdesc[bi].wait()
                    sdesc[bi] = None
            a, b2, c3 = bufs3[bi0], bufs3[bi1], bufs3[bi2]
            d0 = pltpu.async_copy(tables_sh.at[off_v.at[j]], a, gsems[bi0])
            d1 = pltpu.async_copy(tables_sh.at[off_v.at[16 + j]], b2,
                                  gsems[bi1])
            d2 = pltpu.async_copy(tables_sh.at[off_v.at[32 + j]], c3,
                                  gsems[bi2])
            d0.wait()
            d1.wait()
            d2.wait()

            def acc3(r, _, a=a, b2=b2, c3=c3):
                for k in range(8):
                    sl = pl.ds(k * 16, 16)
                    a[r, sl] = a[r, sl] + b2[r, sl] + c3[r, sl]
                return _

            lax.fori_loop(0, 64, acc3, None)
            sdesc[bi0] = pltpu.async_copy(
                a,
                dest_hbm.at[b, pl.ds(pl.multiple_of(s * TPB + j * 64, 8), 64)],
                sa0)
        for bi in range(3):
            if sdesc[bi] is not None:
                sdesc[bi].wait()
        plsc.subcore_barrier()


def _pool1_body(do_gather, c_hbm, idxoff_hbm, inv_hbm, *refs):
    out1 = refs[0]
    rest = refs[1:]
    c = lax.axis_index("c")
    s = lax.axis_index("s")
    for w in range(2):
        b = 2 * c + w
        _pool_wave(do_gather, b, c_hbm, out1, idxoff_hbm, inv_hbm, s, rest)


def _make_pool(do_gather):
    if do_gather:
        one = jax.ShapeDtypeStruct((B, T, 128), jnp.float32)
        mid = [
            pltpu.VMEM((64, 128), jnp.float32),
            pltpu.VMEM((64, 128), jnp.float32),
            pltpu.VMEM((64, 128), jnp.float32),
            pltpu.VMEM((64, 16), jnp.float32),
            pltpu.VMEM((64, 16), jnp.float32),
        ]
    else:
        one = jax.ShapeDtypeStruct((B * TAB, 128), jnp.float32)
        mid = [
            pltpu.VMEM((64, 128), jnp.float32),
            pltpu.VMEM((64, 128), jnp.float32),
            pltpu.VMEM((64, 16), jnp.float32),
            pltpu.VMEM((64, 16), jnp.float32),
        ]
    scratch = (
        [pltpu.VMEM((48, 64), jnp.int32)]
        + mid
        + [
            pltpu.SemaphoreType.DMA,
            pltpu.SemaphoreType.DMA,
            pltpu.SemaphoreType.DMA,
            pltpu.SemaphoreType.DMA,
            pltpu.SemaphoreType.DMA,
            pltpu.VMEM_SHARED((TAB, 128), jnp.float32),
        ]
    )
    return pl.kernel(
        functools.partial(_pool1_body, do_gather),
        out_type=one,
        mesh=_mesh,
        scratch_types=scratch,
        compiler_params=_SC_PARAMS,
    )


_pool_call = _make_pool(True)
_plane_call = _make_pool(False)


RE = 2048
BT = B * T


def _full(shape):
    return pl.BlockSpec(shape, lambda i: (0,) * len(shape))


def _rows(shape):
    return pl.BlockSpec(shape, lambda i: (i,) + (0,) * (len(shape) - 1))


def _dot(a, b):
    return jnp.dot(a, b, preferred_element_type=jnp.float32)


def _embed_body(p_ref, p2_ref, wp, bp, wp2, bp2, wfa, wfb, bf, fp_ref, nc_ref):
    x = p_ref[...]
    x2 = p2_ref[...]
    fp = jnp.maximum(_dot(x, wp[...]) + bp[...], 0.0)
    fp2 = jnp.maximum(_dot(x2, wp2[...]) + bp2[...], 0.0)
    nc = jnp.maximum(_dot(fp, wfa[...]) + _dot(fp2, wfb[...]) + bf[...], 0.0)
    fp_ref[...] = fp
    nc_ref[...] = nc


_embed_call = pl.pallas_call(
    _embed_body,
    grid=(BT // RE,),
    in_specs=[
        _rows((RE, 3)), _rows((RE, 3)),
        _full((3, 2 * HID)), _full((1, 2 * HID)),
        _full((3, 2 * HID)), _full((1, 2 * HID)),
        _full((2 * HID, 2 * HID)), _full((2 * HID, 2 * HID)),
        _full((1, 2 * HID)),
    ],
    out_specs=(_rows((RE, 2 * HID)), _rows((RE, 2 * HID))),
    out_shape=(
        jax.ShapeDtypeStruct((BT, 2 * HID), jnp.float32),
        jax.ShapeDtypeStruct((BT, 2 * HID), jnp.float32),
    ),
)


def _res1_body(x_ref, w0, b0, w1, b1, ws, o_ref):
    x = x_ref[...]
    net = _dot(jnp.maximum(x, 0.0), w0[...]) + b0[...]
    dx = _dot(jnp.maximum(net, 0.0), w1[...]) + b1[...]
    o_ref[...] = _dot(x, ws[...]) + dx


_res1_call = pl.pallas_call(
    _res1_body,
    grid=(BT // RE,),
    in_specs=[
        _rows((RE, 2 * HID)),
        _full((2 * HID, HID)), _full((1, HID)),
        _full((HID, HID)), _full((1, HID)),
        _full((2 * HID, HID)),
    ],
    out_specs=_rows((RE, HID)),
    out_shape=jax.ShapeDtypeStruct((BT, HID), jnp.float32),
)


def _res2_body(has_proj, *refs):
    if has_proj:
        (a_ref, p_ref, w0a, w0b, b0, w1, b1, wsa, wsb, wc, bc, o_ref) = refs
    else:
        (a_ref, p_ref, w0a, w0b, b0, w1, b1, wsa, wsb, o_ref) = refs
    a = a_ref[...]
    p = p_ref[...]
    net = (_dot(jnp.maximum(a, 0.0), w0a[...])
           + _dot(jnp.maximum(p, 0.0), w0b[...]) + b0[...])
    dx = _dot(jnp.maximum(net, 0.0), w1[...]) + b1[...]
    o = _dot(a, wsa[...]) + _dot(p, wsb[...]) + dx
    if has_proj:
        o = _dot(o, wc[...]) + bc[...]
    o_ref[...] = o


def _make_res2(has_proj, mout):
    in_specs = [
        _rows((RE, HID)), _rows((RE, HID)),
        _full((HID, HID)), _full((HID, HID)), _full((1, HID)),
        _full((HID, HID)), _full((1, HID)),
        _full((HID, HID)), _full((HID, HID)),
    ]
    if has_proj:
        in_specs += [_full((HID, mout)), _full((1, mout))]
    return pl.pallas_call(
        functools.partial(_res2_body, has_proj),
        grid=(BT // RE,),
        in_specs=in_specs,
        out_specs=_rows((RE, mout)),
        out_shape=jax.ShapeDtypeStruct((BT, mout), jnp.float32),
    )


_res2_call = _make_res2(False, HID)
_res2p_call = _make_res2(True, C_DIM)


def _res1(x, blk):
    return _res1_call(x, blk['W0'], blk['b0'].reshape(1, -1),
                      blk['W1'], blk['b1'].reshape(1, -1), blk['Ws'])


def _res2(net, pooled, blk, proj=None):
    args = (net, pooled,
            blk['W0'][:HID], blk['W0'][HID:], blk['b0'].reshape(1, -1),
            blk['W1'], blk['b1'].reshape(1, -1),
            blk['Ws'][:HID], blk['Ws'][HID:])
    if proj is None:
        return _res2_call(*args)
    wc, bc = proj
    return _res2p_call(*args, wc, bc.reshape(1, -1))


def kernel(p, p2, params):
    pf = p.reshape(BT, 3)
    p2f = p2.reshape(BT, 3)
    prm = params
    fp, ncorr = _embed_call(
        pf, p2f, prm['Wp'], prm['bp'].reshape(1, -1),
        prm['Wp2'], prm['bp2'].reshape(1, -1),
        prm['Wf'][:2 * HID], prm['Wf'][2 * HID:], prm['bf'].reshape(1, -1))

    pt = jnp.transpose(p, (0, 2, 1)).reshape(B * 3 * 128, 128)
    ones = jnp.ones((64, 16), jnp.float32)
    zc = jnp.zeros((2 * TAB, 16), jnp.float32)
    idxoff, inv = _idx_call(pt, ones, zc)

    blocks_g = prm['blocks']
    blocks_c = prm['blocks_corr']
    net_g = _res1(fp, blocks_g[0])
    net_c = _res1(ncorr, blocks_c[0])
    nblk = len(blocks_g)
    for i in range(1, nblk):
        pooled_g = _pool_call(net_g.reshape(B, T, HID), idxoff, inv)
        pooled_c = _pool_call(net_c.reshape(B, T, HID), idxoff, inv)
        last = i == nblk - 1
        net_g = _res2(net_g, pooled_g.reshape(BT, HID), blocks_g[i],
                      (prm['Wc'], prm['bc']) if last else None)
        net_c = _res2(net_c, pooled_c.reshape(BT, HID), blocks_c[i],
                      (prm['Wc_corr'], prm['bc_corr']) if last else None)

    tabs_c = _plane_call(net_g.reshape(B, T, C_DIM), idxoff, inv)
    tabs_cc = _plane_call(net_c.reshape(B, T, C_DIM), idxoff, inv)

    def to_fea(tabs):
        f = tabs.reshape(B, 3, S, C_DIM)
        f = jnp.transpose(f, (1, 0, 3, 2))
        return f.reshape(3, B, C_DIM, RESO, RESO)

    return jnp.concatenate([to_fea(tabs_c), to_fea(tabs_cc)], axis=0)


# speedup vs baseline: 1.3695x; 1.0461x over previous
import functools

import jax
import jax.numpy as jnp
from jax import lax
from jax.experimental import pallas as pl
from jax.experimental.pallas import tpu as pltpu
from jax.experimental.pallas import tpu_sc as plsc

B = 4
T = 16384
HID = 128
C_DIM = 128
RESO = 64
S = RESO * RESO
PAD = 0.1
NB = 5

NCORE = 2
NSUB = 16
LANES = 16

TPB = T // NSUB
ROWS = T // 128
TAB = 3 * S
DEN = 1.0 + PAD + 1e-3

_mesh = plsc.VectorSubcoreMesh(core_axis_name="c", subcore_axis_name="s")


def _digitize(v):
    u = jnp.clip(v / DEN + 0.5, 0.0, 1.0 - 1e-6)
    return jnp.clip((u * RESO).astype(jnp.int32), 0, RESO - 1)


def _idx_body(pt_hbm, ones_hbm, zc_hbm, idxoff_hbm, inv_hbm,
              px_v, py_v, pz_v, ioff_v, coff_v, ones_v, cnt_v, inv_v, counts_sh):
    c = lax.axis_index("c")
    s = lax.axis_index("s")
    b_local = s // 8
    b = 2 * c + b_local
    base = (s % 8) * 2048

    sh0 = pl.multiple_of(s * 1536, 8)
    pltpu.sync_copy(zc_hbm.at[pl.ds(sh0, 1536)],
                    counts_sh.at[pl.ds(sh0, 1536)])
    pltpu.sync_copy(ones_hbm, ones_v)

    prow = b * 3 * 128 + (s % 8) * 16
    pltpu.sync_copy(pt_hbm.at[pl.ds(pl.multiple_of(prow, 8), 16)], px_v)
    pltpu.sync_copy(pt_hbm.at[pl.ds(pl.multiple_of(prow + 128, 8), 16)], py_v)
    pltpu.sync_copy(pt_hbm.at[pl.ds(pl.multiple_of(prow + 256, 8), 16)], pz_v)

    boff = b_local * TAB

    def step(i, _):
        prw = i // 8
        pcl = pl.ds((i % 8) * 16, 16)
        ix = _digitize(px_v[prw, pcl])
        iy = _digitize(py_v[prw, pcl])
        iz = _digitize(pz_v[prw, pcl])
        row = i // 4
        colsl = pl.ds((i % 4) * 16, 16)
        v0 = ix + 64 * iz
        v1 = ix + 64 * iy + S
        v2 = iy + 64 * iz + 2 * S
        ioff_v[row, colsl] = v0
        ioff_v[32 + row, colsl] = v1
        ioff_v[64 + row, colsl] = v2
        coff_v[row, colsl] = v0 + boff
        coff_v[32 + row, colsl] = v1 + boff
        coff_v[64 + row, colsl] = v2 + boff
        return _

    lax.fori_loop(0, 128, step, None)

    for plane in range(3):
        r0 = pl.multiple_of((b * 3 + plane) * (T // 64) + base // 64, 8)
        pltpu.sync_copy(ioff_v.at[pl.ds(plane * 32, 32)],
                        idxoff_hbm.at[pl.ds(r0, 32)])

    plsc.subcore_barrier()
    for r in range(96):
        pltpu.sync_copy(ones_v, counts_sh.at[coff_v.at[r]], add=True)
    plsc.subcore_barrier()

    pltpu.sync_copy(counts_sh.at[pl.ds(sh0, 1536)], cnt_v)

    def invstep(r, _):
        inv_v[r] = 1.0 / jnp.maximum(cnt_v[r], 1.0)
        return _

    lax.fori_loop(0, 1536, invstep, None)
    pltpu.sync_copy(inv_v, inv_hbm.at[pl.ds(pl.multiple_of(c * 2 * TAB + s * 1536, 8), 1536)])


_SC_PARAMS = pltpu.CompilerParams(use_tc_tiling_on_sc=False)

_idx_call = functools.partial(
    pl.kernel,
    compiler_params=_SC_PARAMS,
    out_type=(
        jax.ShapeDtypeStruct((B * 3 * (T // 64), 64), jnp.int32),
        jax.ShapeDtypeStruct((B * TAB, 16), jnp.float32),
    ),
    mesh=_mesh,
    scratch_types=[
        pltpu.VMEM((16, 128), jnp.float32),
        pltpu.VMEM((16, 128), jnp.float32),
        pltpu.VMEM((16, 128), jnp.float32),
        pltpu.VMEM((96, 64), jnp.int32),
        pltpu.VMEM((96, 64), jnp.int32),
        pltpu.VMEM((64, 16), jnp.float32),
        pltpu.VMEM((1536, 16), jnp.float32),
        pltpu.VMEM((1536, 16), jnp.float32),
        pltpu.VMEM_SHARED((2 * TAB, 16), jnp.float32),
    ],
)(_idx_body)


def _pool_wave(do_gather, b, c_hbm, dest_hbm, idxoff_hbm, inv_hbm, s, refs):
    if do_gather:
        (off_v, ga, gb, gc, inv_c, inv_c2, s0, s1, s2, sa0, sa1,
         tables_sh) = refs
    else:
        (off_v, ga, gb, inv_c, inv_c2, s0, s1, s2, sa0, sa1, tables_sh) = refs
    R64 = T // 64

    def zstep(r, _):
        for k in range(8):
            ga[r, pl.ds(k * 16, 16)] = jnp.zeros((16,), jnp.float32)
        return _

    lax.fori_loop(0, 64, zstep, None)
    zdescs = []
    for j2 in range(12):
        r0 = pl.multiple_of(s * 768 + j2 * 64, 8)
        zdescs.append(pltpu.async_copy(ga, tables_sh.at[pl.ds(r0, 64)], sa0))
    odescs = []
    for plane in range(3):
        ir0 = pl.multiple_of((b * 3 + plane) * R64 + s * 16, 8)
        odescs.append(pltpu.async_copy(idxoff_hbm.at[pl.ds(ir0, 16)],
                                       off_v.at[pl.ds(plane * 16, 16)], sa1))
    for d in zdescs + odescs:
        d.wait()
    plsc.subcore_barrier()

    bufs = (ga, gb)
    lsems = (s0, s1)
    ssems = (sa0, sa1)
    pend_sc = [None, None]

    def chunk_src(j):
        return c_hbm.at[b, pl.ds(pl.multiple_of(s * TPB + j * 64, 8), 64)]

    pend_load = pltpu.async_copy(chunk_src(0), bufs[0], lsems[0])
    for j in range(16):
        pend_load.wait()
        if j < 15:
            nb = (j + 1) % 2
            if pend_sc[nb] is not None:
                for d in pend_sc[nb]:
                    d.wait()
                pend_sc[nb] = None
            pend_load = pltpu.async_copy(chunk_src(j + 1), bufs[nb], lsems[nb])
        pend_sc[j % 2] = [
            pltpu.async_copy(bufs[j % 2],
                             tables_sh.at[off_v.at[plane * 16 + j]],
                             ssems[j % 2], add=True)
            for plane in range(3)
        ]
    for lst in pend_sc:
        if lst is not None:
            for d in lst:
                d.wait()
    plsc.subcore_barrier()

    nbufs = (ga, gb)
    wsems = (s0, s1)
    rsems = (sa0, sa1)
    ibufs = (inv_c, inv_c2)
    wdesc = [None, None]
    rdesc = [None, None]

    def tab_blk(j2):
        return tables_sh.at[pl.ds(pl.multiple_of(s * 768 + j2 * 64, 8), 64)]

    def inv_src(j2):
        gr0 = pl.multiple_of(b * TAB + s * 768 + j2 * 64, 8)
        return inv_hbm.at[pl.ds(gr0, 64)]

    rdesc[0] = pltpu.async_copy(tab_blk(0), nbufs[0], rsems[0])
    ipend = pltpu.async_copy(inv_src(0), ibufs[0], s2)
    for j2 in range(12):
        nb = j2 % 2
        buf = nbufs[nb]
        ib = ibufs[nb]
        rdesc[nb].wait()
        rdesc[nb] = None
        ipend.wait()
        if j2 < 11:
            onb = (j2 + 1) % 2
            if wdesc[onb] is not None:
                wdesc[onb].wait()
                wdesc[onb] = None
            rdesc[onb] = pltpu.async_copy(tab_blk(j2 + 1), nbufs[onb],
                                          rsems[onb])
            ipend = pltpu.async_copy(inv_src(j2 + 1), ibufs[onb], s2)

        def nstep(r, _, buf=buf, ib=ib):
            iv = ib[r]
            for k in range(8):
                sl = pl.ds(k * 16, 16)
                buf[r, sl] = buf[r, sl] * iv
            return _

        lax.fori_loop(0, 64, nstep, None)
        gr0 = pl.multiple_of(b * TAB + s * 768 + j2 * 64, 8)
        if do_gather:
            wdesc[nb] = pltpu.async_copy(buf, tab_blk(j2), wsems[nb])
        else:
            wdesc[nb] = pltpu.async_copy(buf, dest_hbm.at[pl.ds(gr0, 64)],
                                         wsems[nb])
    for i2 in range(2):
        if wdesc[i2] is not None:
            wdesc[i2].wait()
    plsc.subcore_barrier()

    if do_gather:
        bufs3 = (ga, gb, gc)
        gsems = (s0, s1, s2)
        sdesc = [None, None, None]
        for j in range(16):
            bi0 = j % 3
            bi1 = (j + 1) % 3
            bi2 = (j + 2) % 3
            for bi in (bi0, bi1, bi2):
                if sdesc[bi] is not None:
                    sdesc[bi].wait()
                    sdesc[bi] = None
            a, b2, c3 = bufs3[bi0], bufs3[bi1], bufs3[bi2]
            d0 = pltpu.async_copy(tables_sh.at[off_v.at[j]], a, gsems[bi0])
            d1 = pltpu.async_copy(tables_sh.at[off_v.at[16 + j]], b2,
                                  gsems[bi1])
            d2 = pltpu.async_copy(tables_sh.at[off_v.at[32 + j]], c3,
                                  gsems[bi2])
            d0.wait()
            d1.wait()
            d2.wait()

            def acc3(r, _, a=a, b2=b2, c3=c3):
                for k in range(8):
                    sl = pl.ds(k * 16, 16)
                    a[r, sl] = a[r, sl] + b2[r, sl] + c3[r, sl]
                return _

            lax.fori_loop(0, 64, acc3, None)
            sdesc[bi0] = pltpu.async_copy(
                a,
                dest_hbm.at[b, pl.ds(pl.multiple_of(s * TPB + j * 64, 8), 64)],
                sa0)
        for bi in range(3):
            if sdesc[bi] is not None:
                sdesc[bi].wait()
        plsc.subcore_barrier()


def _pool1_body(do_gather, c_hbm, idxoff_hbm, inv_hbm, *refs):
    out1 = refs[0]
    rest = refs[1:]
    c = lax.axis_index("c")
    s = lax.axis_index("s")
    for w in range(2):
        b = 2 * c + w
        _pool_wave(do_gather, b, c_hbm, out1, idxoff_hbm, inv_hbm, s, rest)


def _make_pool(do_gather):
    if do_gather:
        one = jax.ShapeDtypeStruct((B, T, 128), jnp.float32)
        mid = [
            pltpu.VMEM((64, 128), jnp.float32),
            pltpu.VMEM((64, 128), jnp.float32),
            pltpu.VMEM((64, 128), jnp.float32),
            pltpu.VMEM((64, 16), jnp.float32),
            pltpu.VMEM((64, 16), jnp.float32),
        ]
    else:
        one = jax.ShapeDtypeStruct((B * TAB, 128), jnp.float32)
        mid = [
            pltpu.VMEM((64, 128), jnp.float32),
            pltpu.VMEM((64, 128), jnp.float32),
            pltpu.VMEM((64, 16), jnp.float32),
            pltpu.VMEM((64, 16), jnp.float32),
        ]
    scratch = (
        [pltpu.VMEM((48, 64), jnp.int32)]
        + mid
        + [
            pltpu.SemaphoreType.DMA,
            pltpu.SemaphoreType.DMA,
            pltpu.SemaphoreType.DMA,
            pltpu.SemaphoreType.DMA,
            pltpu.SemaphoreType.DMA,
            pltpu.VMEM_SHARED((TAB, 128), jnp.float32),
        ]
    )
    return pl.kernel(
        functools.partial(_pool1_body, do_gather),
        out_type=one,
        mesh=_mesh,
        scratch_types=scratch,
        compiler_params=_SC_PARAMS,
    )


_pool_call = _make_pool(True)
_plane_call = _make_pool(False)


RE = 2048
BT = B * T


def _full(shape):
    return pl.BlockSpec(shape, lambda i: (0,) * len(shape))


def _rows(shape):
    return pl.BlockSpec(shape, lambda i: (i,) + (0,) * (len(shape) - 1))


def _dot(a, b):
    return jnp.dot(a, b, preferred_element_type=jnp.float32)


def _embed_body(p_ref, p2_ref, wp, bp, wp2, bp2, wfa, wfb, bf, fp_ref, nc_ref):
    x = p_ref[...]
    x2 = p2_ref[...]
    fp = jnp.maximum(_dot(x, wp[...]) + bp[...], 0.0)
    fp2 = jnp.maximum(_dot(x2, wp2[...]) + bp2[...], 0.0)
    nc = jnp.maximum(_dot(fp, wfa[...]) + _dot(fp2, wfb[...]) + bf[...], 0.0)
    fp_ref[...] = fp
    nc_ref[...] = nc


_embed_call = pl.pallas_call(
    _embed_body,
    grid=(BT // RE,),
    in_specs=[
        _rows((RE, 3)), _rows((RE, 3)),
        _full((3, 2 * HID)), _full((1, 2 * HID)),
        _full((3, 2 * HID)), _full((1, 2 * HID)),
        _full((2 * HID, 2 * HID)), _full((2 * HID, 2 * HID)),
        _full((1, 2 * HID)),
    ],
    out_specs=(_rows((RE, 2 * HID)), _rows((RE, 2 * HID))),
    out_shape=(
        jax.ShapeDtypeStruct((BT, 2 * HID), jnp.float32),
        jax.ShapeDtypeStruct((BT, 2 * HID), jnp.float32),
    ),
)


def _res1_body(x_ref, w0, b0, w1, b1, ws, o_ref):
    x = x_ref[...]
    net = _dot(jnp.maximum(x, 0.0), w0[...]) + b0[...]
    dx = _dot(jnp.maximum(net, 0.0), w1[...]) + b1[...]
    o_ref[...] = _dot(x, ws[...]) + dx


_res1_call = pl.pallas_call(
    _res1_body,
    grid=(BT // RE,),
    in_specs=[
        _rows((RE, 2 * HID)),
        _full((2 * HID, HID)), _full((1, HID)),
        _full((HID, HID)), _full((1, HID)),
        _full((2 * HID, HID)),
    ],
    out_specs=_rows((RE, HID)),
    out_shape=jax.ShapeDtypeStruct((BT, HID), jnp.float32),
)


def _res2_body(has_proj, *refs):
    if has_proj:
        (a_ref, p_ref, w0a, w0b, b0, w1, b1, wsa, wsb, wc, bc, o_ref) = refs
    else:
        (a_ref, p_ref, w0a, w0b, b0, w1, b1, wsa, wsb, o_ref) = refs
    a = a_ref[...]
    p = p_ref[...]
    net = (_dot(jnp.maximum(a, 0.0), w0a[...])
           + _dot(jnp.maximum(p, 0.0), w0b[...]) + b0[...])
    dx = _dot(jnp.maximum(net, 0.0), w1[...]) + b1[...]
    o = _dot(a, wsa[...]) + _dot(p, wsb[...]) + dx
    if has_proj:
        o = _dot(o, wc[...]) + bc[...]
    o_ref[...] = o


def _make_res2(has_proj, mout):
    in_specs = [
        _rows((RE, HID)), _rows((RE, HID)),
        _full((HID, HID)), _full((HID, HID)), _full((1, HID)),
        _full((HID, HID)), _full((1, HID)),
        _full((HID, HID)), _full((HID, HID)),
    ]
    if has_proj:
        in_specs += [_full((HID, mout)), _full((1, mout))]
    return pl.pallas_call(
        functools.partial(_res2_body, has_proj),
        grid=(BT // RE,),
        in_specs=in_specs,
        out_specs=_rows((RE, mout)),
        out_shape=jax.ShapeDtypeStruct((BT, mout), jnp.float32),
    )


_res2_call = _make_res2(False, HID)
_res2p_call = _make_res2(True, C_DIM)


def _res1(x, blk):
    return _res1_call(x, blk['W0'], blk['b0'].reshape(1, -1),
                      blk['W1'], blk['b1'].reshape(1, -1), blk['Ws'])


def _res2(net, pooled, blk, proj=None):
    args = (net, pooled,
            blk['W0'][:HID], blk['W0'][HID:], blk['b0'].reshape(1, -1),
            blk['W1'], blk['b1'].reshape(1, -1),
            blk['Ws'][:HID], blk['Ws'][HID:])
    if proj is None:
        return _res2_call(*args)
    wc, bc = proj
    return _res2p_call(*args, wc, bc.reshape(1, -1))


def kernel(p, p2, params):
    pf = p.reshape(BT, 3)
    p2f = p2.reshape(BT, 3)
    prm = params
    fp, ncorr = _embed_call(
        pf, p2f, prm['Wp'], prm['bp'].reshape(1, -1),
        prm['Wp2'], prm['bp2'].reshape(1, -1),
        prm['Wf'][:2 * HID], prm['Wf'][2 * HID:], prm['bf'].reshape(1, -1))

    pt = jnp.transpose(p, (0, 2, 1)).reshape(B * 3 * 128, 128)
    ones = jnp.ones((64, 16), jnp.float32)
    zc = jnp.zeros((2 * TAB, 16), jnp.float32)
    idxoff, inv = _idx_call(pt, ones, zc)

    blocks_g = prm['blocks']
    blocks_c = prm['blocks_corr']
    net_g = _res1(fp, blocks_g[0])
    net_c = _res1(ncorr, blocks_c[0])
    nblk = len(blocks_g)
    for i in range(1, nblk):
        pooled_g = _pool_call(net_g.reshape(B, T, HID), idxoff, inv)
        pooled_c = _pool_call(net_c.reshape(B, T, HID), idxoff, inv)
        last = i == nblk - 1
        net_g = _res2(net_g, pooled_g.reshape(BT, HID), blocks_g[i],
                      (prm['Wc'], prm['bc']) if last else None)
        net_c = _res2(net_c, pooled_c.reshape(BT, HID), blocks_c[i],
                      (prm['Wc_corr'], prm['bc_corr']) if last else None)

    tabs_c = _plane_call(net_g.reshape(B, T, C_DIM), idxoff, inv)
    tabs_cc = _plane_call(net_c.reshape(B, T, C_DIM), idxoff, inv)

    def to_fea(tabs):
        f = tabs.reshape(B, 3, S, C_DIM)
        f = jnp.transpose(f, (1, 0, 3, 2))
        return f.reshape(3, B, C_DIM, RESO, RESO)

    return jnp.concatenate([to_fea(tabs_c), to_fea(tabs_cc)], axis=0)
